# baseline (speedup 1.0000x reference)
import jax
import jax.numpy as jnp
from jax.experimental import pallas as pl
from jax.experimental.pallas import tpu as pltpu

_N = 4096
_HALF_ROWS = 2048
_FEAT = 32
_BM = 128
_NBLK = _N // _BM
_HALF = _HALF_ROWS // _BM


def _gcn_body(a1_ref, a2_ref, e1_ref, e2_ref, w_ref, d_ref, p_ref, m_ref):
    m = pl.program_id(0)
    e1 = e1_ref[...]
    e2 = e2_ref[...]
    x1 = jnp.dot(a1_ref[...].reshape(2 * _BM, _N), e1, preferred_element_type=jnp.float32)
    x2 = jnp.dot(a2_ref[...].reshape(2 * _BM, _N), e2, preferred_element_type=jnp.float32)
    t1 = jnp.maximum(x1[:_BM], 0.0) + jnp.maximum(x1[_BM:], 0.0)
    t2 = jnp.maximum(x2[:_BM], 0.0) + jnp.maximum(x2[_BM:], 0.0)
    t1 = t1 + t1
    t2 = t2 + t2

    @pl.when(m < _HALF)
    def _():
        d_ref[...] = t1
        p_ref[...] = t2

    @pl.when(m >= _HALF)
    def _():
        w = w_ref[0]
        m_ref[...] = w * t1 + (1.0 - w) * t2


def kernel(adj1, adj2, dEmbed, mEmbed, pEmbed, inter):
    e1 = jnp.concatenate([dEmbed, mEmbed], axis=0)
    e2 = jnp.concatenate([pEmbed, mEmbed], axis=0)
    d_out, p_out, m_out = pl.pallas_call(
        _gcn_body,
        grid=(_NBLK,),
        in_specs=[
            pl.BlockSpec((2, _BM, _N), lambda m: (0, m, 0)),
            pl.BlockSpec((2, _BM, _N), lambda m: (0, m, 0)),
            pl.BlockSpec((_N, _FEAT), lambda m: (0, 0)),
            pl.BlockSpec((_N, _FEAT), lambda m: (0, 0)),
            pl.BlockSpec(memory_space=pltpu.SMEM),
        ],
        out_specs=[
            pl.BlockSpec((_BM, _FEAT), lambda m: (jnp.minimum(m, _HALF - 1), 0)),
            pl.BlockSpec((_BM, _FEAT), lambda m: (jnp.minimum(m, _HALF - 1), 0)),
            pl.BlockSpec((_BM, _FEAT), lambda m: (jnp.maximum(m - _HALF, 0), 0)),
        ],
        out_shape=[
            jax.ShapeDtypeStruct((_HALF_ROWS, _FEAT), jnp.float32),
            jax.ShapeDtypeStruct((_HALF_ROWS, _FEAT), jnp.float32),
            jax.ShapeDtypeStruct((_HALF_ROWS, _FEAT), jnp.float32),
        ],
        compiler_params=pltpu.CompilerParams(dimension_semantics=("arbitrary",)),
    )(adj1, adj2, e1, e2, inter)
    return (m_out, d_out, p_out)


# speedup vs baseline: 1.0208x; 1.0208x over previous
import jax
import jax.numpy as jnp
from jax.experimental import pallas as pl
from jax.experimental.pallas import tpu as pltpu

_N = 4096
_HALF_ROWS = 2048
_FEAT = 32
_BM = 128
_NBLK = _N // _BM
_HALF = _HALF_ROWS // _BM


def _gcn_body(a1_ref, a2_ref, e1_ref, e2_ref, w_ref, d_ref, p_ref, m_ref):
    m = pl.program_id(0)
    e1 = e1_ref[...]
    e2 = e2_ref[...]
    t1 = jnp.maximum(jnp.dot(a1_ref[0], e1, preferred_element_type=jnp.float32), 0.0)
    t1 = t1 + jnp.maximum(jnp.dot(a1_ref[1], e1, preferred_element_type=jnp.float32), 0.0)
    t2 = jnp.maximum(jnp.dot(a2_ref[0], e2, preferred_element_type=jnp.float32), 0.0)
    t2 = t2 + jnp.maximum(jnp.dot(a2_ref[1], e2, preferred_element_type=jnp.float32), 0.0)
    t1 = t1 + t1
    t2 = t2 + t2

    @pl.when(m < _HALF)
    def _():
        d_ref[...] = t1
        p_ref[...] = t2

    @pl.when(m >= _HALF)
    def _():
        w = w_ref[0]
        m_ref[...] = w * t1 + (1.0 - w) * t2


def kernel(adj1, adj2, dEmbed, mEmbed, pEmbed, inter):
    e1 = jnp.concatenate([dEmbed, mEmbed], axis=0)
    e2 = jnp.concatenate([pEmbed, mEmbed], axis=0)
    d_out, p_out, m_out = pl.pallas_call(
        _gcn_body,
        grid=(_NBLK,),
        in_specs=[
            pl.BlockSpec((2, _BM, _N), lambda m: (0, m, 0)),
            pl.BlockSpec((2, _BM, _N), lambda m: (0, m, 0)),
            pl.BlockSpec((_N, _FEAT), lambda m: (0, 0)),
            pl.BlockSpec((_N, _FEAT), lambda m: (0, 0)),
            pl.BlockSpec(memory_space=pltpu.SMEM),
        ],
        out_specs=[
            pl.BlockSpec((_BM, _FEAT), lambda m: (jnp.minimum(m, _HALF - 1), 0)),
            pl.BlockSpec((_BM, _FEAT), lambda m: (jnp.minimum(m, _HALF - 1), 0)),
            pl.BlockSpec((_BM, _FEAT), lambda m: (jnp.maximum(m - _HALF, 0), 0)),
        ],
        out_shape=[
            jax.ShapeDtypeStruct((_HALF_ROWS, _FEAT), jnp.float32),
            jax.ShapeDtypeStruct((_HALF_ROWS, _FEAT), jnp.float32),
            jax.ShapeDtypeStruct((_HALF_ROWS, _FEAT), jnp.float32),
        ],
        compiler_params=pltpu.CompilerParams(dimension_semantics=("arbitrary",)),
    )(adj1, adj2, e1, e2, inter)
    return (m_out, d_out, p_out)
